# Initial kernel scaffold; baseline (speedup 1.0000x reference)
import jax
import jax.numpy as jnp
from jax.experimental import pallas as pl


def kernel(input, target):
    raise NotImplementedError("write your pallas kernel here")


# speedup vs baseline: 48.6287x; 48.6287x over previous
import functools

import jax
import jax.numpy as jnp
from jax import lax
from jax.experimental import pallas as pl
from jax.experimental.pallas import tpu as pltpu
from jax.experimental.pallas import tpu_sc as plsc

C = 19
B = 4
HW = 512 * 512
N_PIX = B * HW

NB_SUB = 9
EXP_LO = 113
N_EXP = 20
NB = N_EXP << NB_SUB
OFFSET = EXP_LO << NB_SUB
NH = 2 * NB

N_TILES = 32
TILES_PER_BATCH = N_TILES // B
PIX = HW // TILES_PER_BATCH
LANES = 16
UNROLL = 4


def _hist_body(x_hbm, t_hbm, out_hbm, xbuf, tbuf, hist):
    wid = lax.axis_index("s") * 2 + lax.axis_index("c")
    bi = wid // TILES_PER_BATCH
    hw0 = (wid % TILES_PER_BATCH) * PIX

    pltpu.sync_copy(t_hbm.at[bi, pl.ds(hw0, PIX)], tbuf)

    zeros16 = jnp.zeros((LANES,), jnp.float32)
    ones16 = jnp.ones((LANES,), jnp.float32)

    def class_step(c, carry):
        pltpu.sync_copy(x_hbm.at[bi, c, pl.ds(hw0, PIX)], xbuf)

        def zero_step(j, carry2):
            base = j * (LANES * 8)
            for k in range(8):
                hist[pl.ds(base + k * LANES, LANES)] = zeros16
            return carry2

        lax.fori_loop(0, NH // (LANES * 8), zero_step, 0)

        def pix_step(i, carry2):
            base = i * (LANES * UNROLL)
            for k in range(UNROLL):
                off = base + k * LANES
                x = xbuf[pl.ds(off, LANES)]
                t = tbuf[pl.ds(off, LANES)]
                m = t == c
                fg = jnp.where(m, 1.0, 0.0).astype(jnp.float32)
                err = jnp.abs(x - fg)
                bits = lax.bitcast_convert_type(err, jnp.int32)
                bn = lax.shift_right_logical(bits, 14) - OFFSET
                bn = jnp.minimum(jnp.maximum(bn, 0), NB - 1)
                idx = bn + jnp.where(m, NB, 0)
                plsc.addupdate_scatter(hist, [idx], ones16)
            return carry2

        lax.fori_loop(0, PIX // (LANES * UNROLL), pix_step, 0)

        pltpu.sync_copy(hist, out_hbm.at[wid, c])
        return carry

    lax.fori_loop(0, C, class_step, 0)


def _loss_body(hist_ref, out_ref, acc_ref):
    i = pl.program_id(0)

    @pl.when(i == 0)
    def _():
        acc_ref[...] = jnp.zeros_like(acc_ref)

    acc_ref[...] += hist_ref[0]

    @pl.when(i == N_TILES - 1)
    def _():
        H = acc_ref[...]
        bg = H[:, :NB]
        fgc = H[:, NB:]
        cnt = bg + fgc
        G = jnp.sum(fgc, axis=1, keepdims=True)
        total = jnp.sum(cnt, axis=1, keepdims=True)

        def cumsum_lanes(x):
            s = 1
            while s < NB:
                shifted = jnp.concatenate(
                    [jnp.zeros((C, s), jnp.float32), x[:, :-s]], axis=1)
                x = x + shifted
                s *= 2
            return x

        incl = cumsum_lanes(cnt)
        F_incl = cumsum_lanes(fgc)
        n_ge = total - incl + cnt
        F_ge = G - F_incl + fgc
        n_gt = n_ge - cnt
        F_gt = F_ge - fgc

        def jac(n, F):
            den = jnp.maximum(G + n - F, 1.0)
            return 1.0 - (G - F) / den

        bb = lax.broadcasted_iota(jnp.int32, (C, NB), 1)
        rep_bits = ((bb + OFFSET) << 14) + (1 << 13)
        rep = lax.bitcast_convert_type(rep_bits, jnp.float32)

        contrib = rep * (jac(n_ge, F_ge) - jac(n_gt, F_gt))
        losses = jnp.sum(contrib, axis=1, keepdims=True)
        present = (G > 0).astype(jnp.float32)
        num = jnp.sum(losses * present, keepdims=True)
        den = jnp.sum(present, keepdims=True)
        out_ref[...] = num / den


def kernel(input, target):
    x = input.reshape(B, C, HW)
    t = target.reshape(B, HW).astype(jnp.int32)

    mesh = plsc.VectorSubcoreMesh(core_axis_name="c", subcore_axis_name="s")
    hist_kernel = functools.partial(
        pl.kernel,
        mesh=mesh,
        compiler_params=pltpu.CompilerParams(needs_layout_passes=False),
        out_type=jax.ShapeDtypeStruct((N_TILES, C, NH), jnp.float32),
        scratch_types=[
            pltpu.VMEM((PIX,), jnp.float32),
            pltpu.VMEM((PIX,), jnp.int32),
            pltpu.VMEM((NH,), jnp.float32),
        ],
    )(_hist_body)
    partials = hist_kernel(x, t)

    loss = pl.pallas_call(
        _loss_body,
        grid=(N_TILES,),
        in_specs=[pl.BlockSpec((1, C, NH), lambda i: (i, 0, 0))],
        out_specs=pl.BlockSpec((1, 1), lambda i: (0, 0)),
        out_shape=jax.ShapeDtypeStruct((1, 1), jnp.float32),
        scratch_shapes=[pltpu.VMEM((C, NH), jnp.float32)],
    )(partials)
    return loss[0, 0]
